# Initial kernel scaffold; baseline (speedup 1.0000x reference)
import jax
import jax.numpy as jnp
from jax import lax
from jax.experimental import pallas as pl
from jax.experimental.pallas import tpu as pltpu
from jax.experimental.pallas import tpu_sc as plsc

N = 10000
E = 320000
D_IN = 128
D_HID = 256
N_CLASSES = 41
CP = 48

NC = 2
NS = 16
NW = NC * NS

CHUNK = 128
NCHUNK = 80
E_PAD = NW * NCHUNK * CHUNK
N_PAD = 10240
ROWS_PT = N_PAD // NS
RB = 512


def _sc_agg(width, with_count):
  mesh = plsc.VectorSubcoreMesh(
      core_axis_name="c", subcore_axis_name="s", num_cores=NC, num_subcores=NS)

  out_type = [jax.ShapeDtypeStruct((NC, N_PAD, width), jnp.float32)]
  scratch = [
      pltpu.VMEM((NCHUNK, CHUNK), jnp.int32),
      pltpu.VMEM((NCHUNK, CHUNK), jnp.int32),
      pltpu.VMEM((CHUNK, width), jnp.float32),
      pltpu.VMEM_SHARED((N_PAD, width), jnp.float32),
      pltpu.SemaphoreType.DMA,
  ]
  if with_count:
    out_type.append(jax.ShapeDtypeStruct((NC, N_PAD, 16), jnp.float32))
    scratch += [
        pltpu.VMEM((CHUNK, 16), jnp.float32),
        pltpu.VMEM((CHUNK, 16), jnp.float32),
        pltpu.VMEM_SHARED((N_PAD, 16), jnp.float32),
    ]

  WL = width // 16

  def body(feat_hbm, srcs_hbm, dsts_hbm, *rest):
    if with_count:
      (sum_hbm, cnt_hbm, sidx, didx, rows, acc_sh, sem,
       ones_v, z16, cnt_sh) = rest
    else:
      sum_hbm, sidx, didx, rows, acc_sh, sem = rest

    c = lax.axis_index("c")
    s = lax.axis_index("s")
    tile = c * NS + s

    pltpu.sync_copy(srcs_hbm.at[tile], sidx)
    pltpu.sync_copy(dsts_hbm.at[tile], didx)

    def zrow(i, _):
      r = i // WL
      q = (i % WL) * 16
      rows[r, pl.ds(q, 16)] = jnp.zeros((16,), jnp.float32)
      return _
    lax.fori_loop(0, CHUNK * WL, zrow, None)
    if with_count:
      def f16(r, _):
        ones_v[r] = jnp.ones((16,), jnp.float32)
        z16[r] = jnp.zeros((16,), jnp.float32)
        return _
      lax.fori_loop(0, CHUNK, f16, None)

    row0 = s * ROWS_PT
    for k in range(ROWS_PT // CHUNK):
      pltpu.sync_copy(rows, acc_sh.at[pl.ds(row0 + k * CHUNK, CHUNK)])
      if with_count:
        pltpu.sync_copy(z16, cnt_sh.at[pl.ds(row0 + k * CHUNK, CHUNK)])
    plsc.subcore_barrier()

    def step(j, _):
      pltpu.async_copy(feat_hbm.at[sidx.at[j]], rows, sem).wait()
      pltpu.sync_copy(rows, acc_sh.at[didx.at[j]], add=True)
      if with_count:
        pltpu.sync_copy(ones_v, cnt_sh.at[didx.at[j]], add=True)
      return _
    lax.fori_loop(0, NCHUNK, step, None)
    plsc.subcore_barrier()

    pltpu.sync_copy(acc_sh.at[pl.ds(row0, ROWS_PT)],
                    sum_hbm.at[c, pl.ds(row0, ROWS_PT)])
    if with_count:
      pltpu.sync_copy(cnt_sh.at[pl.ds(row0, ROWS_PT)],
                      cnt_hbm.at[c, pl.ds(row0, ROWS_PT)])

  return pl.kernel(body, out_type=out_type, mesh=mesh, scratch_types=scratch)


_sc_agg_feat = _sc_agg(D_IN, with_count=True)
_sc_agg_proj = _sc_agg(CP, with_count=False)


def _tc_fused(xp, sum0, sum1, cnt0, cnt1, ws1, wn1, b1, ws2, wn2, b2):
  def body(x_r, s0_r, s1_r, c0_r, c1_r, ws1_r, wn1_r, b1_r, ws2_r, wn2_r,
           b2_r, p_r, s_r):
    deg = c0_r[:, 0:1] + c1_r[:, 0:1]
    dinv = 1.0 / jnp.maximum(deg, 1.0)
    a = (s0_r[...] + s1_r[...]) * dinv
    h = (jnp.dot(x_r[...], ws1_r[...], preferred_element_type=jnp.float32)
         + jnp.dot(a, wn1_r[...], preferred_element_type=jnp.float32)
         + b1_r[...])
    h = jnp.maximum(h, 0.0)
    p_r[...] = jnp.dot(h, wn2_r[...], preferred_element_type=jnp.float32)
    s_r[...] = (jnp.dot(h, ws2_r[...], preferred_element_type=jnp.float32)
                + b2_r[...])

  row = lambda i: (i, 0)
  fix = lambda i: (0, 0)
  return pl.pallas_call(
      body,
      grid=(N_PAD // RB,),
      in_specs=[
          pl.BlockSpec((RB, D_IN), row),
          pl.BlockSpec((RB, D_IN), row),
          pl.BlockSpec((RB, D_IN), row),
          pl.BlockSpec((RB, 16), row),
          pl.BlockSpec((RB, 16), row),
          pl.BlockSpec((D_IN, D_HID), fix),
          pl.BlockSpec((D_IN, D_HID), fix),
          pl.BlockSpec((1, D_HID), fix),
          pl.BlockSpec((D_HID, CP), fix),
          pl.BlockSpec((D_HID, CP), fix),
          pl.BlockSpec((1, CP), fix),
      ],
      out_specs=[pl.BlockSpec((RB, CP), row), pl.BlockSpec((RB, CP), row)],
      out_shape=[jax.ShapeDtypeStruct((N_PAD, CP), jnp.float32)] * 2,
  )(xp, sum0, sum1, cnt0, cnt1, ws1, wn1, b1, ws2, wn2, b2)


def _tc_final(svec, a0, a1, cnt0, cnt1):
  def body(s_r, a0_r, a1_r, c0_r, c1_r, o_r):
    deg = c0_r[:, 0:1] + c1_r[:, 0:1]
    dinv = 1.0 / jnp.maximum(deg, 1.0)
    o_r[...] = s_r[...] + (a0_r[...] + a1_r[...]) * dinv

  row = lambda i: (i, 0)
  return pl.pallas_call(
      body,
      grid=(N_PAD // RB,),
      in_specs=[
          pl.BlockSpec((RB, CP), row),
          pl.BlockSpec((RB, CP), row),
          pl.BlockSpec((RB, CP), row),
          pl.BlockSpec((RB, 16), row),
          pl.BlockSpec((RB, 16), row),
      ],
      out_specs=pl.BlockSpec((RB, CP), row),
      out_shape=jax.ShapeDtypeStruct((N_PAD, CP), jnp.float32),
  )(svec, a0, a1, cnt0, cnt1)


@jax.jit
def kernel(x, edge_index, W_self1, W_neigh1, b1, W_self2, W_neigh2, b2):
  src = edge_index[0]
  dst = edge_index[1]
  pad_e = E_PAD - E
  srcp = jnp.concatenate(
      [src, jnp.zeros((pad_e,), jnp.int32)]).reshape(NW, NCHUNK, CHUNK)
  dstp = jnp.concatenate(
      [dst, jnp.full((pad_e,), N_PAD - 1, jnp.int32)]).reshape(NW, NCHUNK, CHUNK)
  xp = jnp.pad(x, ((0, N_PAD - N), (0, 0)))
  ws2 = jnp.pad(W_self2, ((0, 0), (0, CP - N_CLASSES)))
  wn2 = jnp.pad(W_neigh2, ((0, 0), (0, CP - N_CLASSES)))
  b2p = jnp.pad(b2, (0, CP - N_CLASSES)).reshape(1, CP)

  sums, cnts = _sc_agg_feat(xp, srcp, dstp)
  p, svec = _tc_fused(xp, sums[0], sums[1], cnts[0], cnts[1],
                      W_self1, W_neigh1, b1.reshape(1, D_HID), ws2, wn2, b2p)
  sums2 = _sc_agg_proj(p, srcp, dstp)
  if isinstance(sums2, (list, tuple)):
    sums2 = sums2[0]
  out = _tc_final(svec, sums2[0], sums2[1], cnts[0], cnts[1])
  return out[:N, :N_CLASSES]


# speedup vs baseline: 7.2129x; 7.2129x over previous
import jax
import jax.numpy as jnp
from jax import lax
from jax.experimental import pallas as pl
from jax.experimental.pallas import tpu as pltpu
from jax.experimental.pallas import tpu_sc as plsc

N = 10000
E = 320000
D_IN = 128
D_HID = 256
N_CLASSES = 41
CP = 48
F1 = 144

NC = 2
NS = 16
NW = NC * NS

CHUNK = 112
NCHUNK = 90
E_PAD = NW * NCHUNK * CHUNK
N_PAD = 10240
ROWS_PT = N_PAD // NS
RB = 512


def _sc_agg(width):
  mesh = plsc.VectorSubcoreMesh(
      core_axis_name="c", subcore_axis_name="s", num_cores=NC, num_subcores=NS)

  out_type = jax.ShapeDtypeStruct((NC, N_PAD, width), jnp.float32)
  scratch = [
      pltpu.VMEM((NCHUNK, CHUNK), jnp.int32),
      pltpu.VMEM((NCHUNK, CHUNK), jnp.int32),
      pltpu.VMEM((CHUNK, width), jnp.float32),
      pltpu.VMEM_SHARED((N_PAD, width), jnp.float32),
      pltpu.SemaphoreType.DMA,
  ]
  WL = width // 16

  def body(feat_hbm, srcs_hbm, dsts_hbm, sum_hbm, sidx, didx, rows, acc_sh,
           sem):
    c = lax.axis_index("c")
    s = lax.axis_index("s")
    tile = c * NS + s

    pltpu.sync_copy(srcs_hbm.at[tile], sidx)
    pltpu.sync_copy(dsts_hbm.at[tile], didx)

    def zrow(i, _):
      r = i // WL
      q = (i % WL) * 16
      rows[r, pl.ds(q, 16)] = jnp.zeros((16,), jnp.float32)
      return _
    lax.fori_loop(0, CHUNK * WL, zrow, None)

    row0 = s * ROWS_PT
    full, part = divmod(ROWS_PT, CHUNK)
    for k in range(full):
      pltpu.sync_copy(rows, acc_sh.at[pl.ds(row0 + k * CHUNK, CHUNK)])
    if part:
      pltpu.sync_copy(rows.at[pl.ds(0, part)],
                      acc_sh.at[pl.ds(row0 + full * CHUNK, part)])
    plsc.subcore_barrier()

    def step(j, _):
      pltpu.async_copy(feat_hbm.at[sidx.at[j]], rows, sem).wait()
      pltpu.sync_copy(rows, acc_sh.at[didx.at[j]], add=True)
      return _
    lax.fori_loop(0, NCHUNK, step, None)
    plsc.subcore_barrier()

    pltpu.sync_copy(acc_sh.at[pl.ds(row0, ROWS_PT)],
                    sum_hbm.at[c, pl.ds(row0, ROWS_PT)])

  return pl.kernel(
      body, out_type=out_type, mesh=mesh, scratch_types=scratch,
      compiler_params=pltpu.CompilerParams(use_tc_tiling_on_sc=False))


_sc_agg_feat = _sc_agg(F1)
_sc_agg_proj = _sc_agg(CP)


def _tc_fused(xp, sum0, sum1, ws1, wn1, b1, ws2, wn2, b2):
  def body(x_r, s0_r, s1_r, ws1_r, wn1_r, b1_r, ws2_r, wn2_r, b2_r, p_r, s_r):
    deg = s0_r[:, D_IN:D_IN + 1] + s1_r[:, D_IN:D_IN + 1]
    dinv = 1.0 / jnp.maximum(deg, 1.0)
    a = (s0_r[:, :D_IN] + s1_r[:, :D_IN]) * dinv
    h = (jnp.dot(x_r[...], ws1_r[...], preferred_element_type=jnp.float32)
         + jnp.dot(a, wn1_r[...], preferred_element_type=jnp.float32)
         + b1_r[...])
    h = jnp.maximum(h, 0.0)
    p_r[...] = jnp.dot(h, wn2_r[...], preferred_element_type=jnp.float32)
    s_r[...] = (jnp.dot(h, ws2_r[...], preferred_element_type=jnp.float32)
                + b2_r[...])

  row = lambda i: (i, 0)
  fix = lambda i: (0, 0)
  return pl.pallas_call(
      body,
      grid=(N_PAD // RB,),
      in_specs=[
          pl.BlockSpec((RB, D_IN), row),
          pl.BlockSpec((RB, F1), row),
          pl.BlockSpec((RB, F1), row),
          pl.BlockSpec((D_IN, D_HID), fix),
          pl.BlockSpec((D_IN, D_HID), fix),
          pl.BlockSpec((1, D_HID), fix),
          pl.BlockSpec((D_HID, CP), fix),
          pl.BlockSpec((D_HID, CP), fix),
          pl.BlockSpec((1, CP), fix),
      ],
      out_specs=[pl.BlockSpec((RB, CP), row), pl.BlockSpec((RB, CP), row)],
      out_shape=[jax.ShapeDtypeStruct((N_PAD, CP), jnp.float32)] * 2,
  )(xp, sum0, sum1, ws1, wn1, b1, ws2, wn2, b2)


def _tc_final(svec, a0, a1, d0, d1):
  def body(s_r, a0_r, a1_r, d0_r, d1_r, o_r):
    deg = d0_r[:, 0:1] + d1_r[:, 0:1]
    dinv = 1.0 / jnp.maximum(deg, 1.0)
    o_r[...] = s_r[...] + (a0_r[...] + a1_r[...]) * dinv

  row = lambda i: (i, 0)
  return pl.pallas_call(
      body,
      grid=(N_PAD // RB,),
      in_specs=[
          pl.BlockSpec((RB, CP), row),
          pl.BlockSpec((RB, CP), row),
          pl.BlockSpec((RB, CP), row),
          pl.BlockSpec((RB, 16), row),
          pl.BlockSpec((RB, 16), row),
      ],
      out_specs=pl.BlockSpec((RB, CP), row),
      out_shape=jax.ShapeDtypeStruct((N_PAD, CP), jnp.float32),
  )(svec, a0, a1, d0, d1)


@jax.jit
def kernel(x, edge_index, W_self1, W_neigh1, b1, W_self2, W_neigh2, b2):
  src = edge_index[0]
  dst = edge_index[1]
  pad_e = E_PAD - E
  srcp = jnp.concatenate(
      [src, jnp.zeros((pad_e,), jnp.int32)]).reshape(NW, NCHUNK, CHUNK)
  dstp = jnp.concatenate(
      [dst, jnp.full((pad_e,), N_PAD - 1, jnp.int32)]).reshape(NW, NCHUNK, CHUNK)
  xp = jnp.pad(x, ((0, N_PAD - N), (0, 0)))
  xa = jnp.pad(jnp.concatenate(
      [xp, jnp.ones((N_PAD, 1), jnp.float32)], axis=1),
      ((0, 0), (0, F1 - D_IN - 1)))
  ws2 = jnp.pad(W_self2, ((0, 0), (0, CP - N_CLASSES)))
  wn2 = jnp.pad(W_neigh2, ((0, 0), (0, CP - N_CLASSES)))
  b2p = jnp.pad(b2, (0, CP - N_CLASSES)).reshape(1, CP)

  sums = _sc_agg_feat(xa, srcp, dstp)
  p, svec = _tc_fused(xp, sums[0], sums[1],
                      W_self1, W_neigh1, b1.reshape(1, D_HID), ws2, wn2, b2p)
  sums2 = _sc_agg_proj(p, srcp, dstp)
  out = _tc_final(svec, sums2[0], sums2[1],
                  sums[0, :, D_IN:D_IN + 16], sums[1, :, D_IN:D_IN + 16])
  return out[:N, :N_CLASSES]


# speedup vs baseline: 8.4372x; 1.1697x over previous
import jax
import jax.numpy as jnp
from jax import lax
from jax.experimental import pallas as pl
from jax.experimental.pallas import tpu as pltpu
from jax.experimental.pallas import tpu_sc as plsc

N = 10000
E = 320000
D_IN = 128
D_HID = 256
N_CLASSES = 41
CP = 48
F1 = 144

NC = 2
NS = 16
NW = NC * NS

CHUNK = 56
NCHUNK = 180
NPAIR = NCHUNK // 2
E_PAD = NW * NCHUNK * CHUNK
N_PAD = 10240
ROWS_PT = N_PAD // NS
RB = 512


def _sc_agg(width):
  mesh = plsc.VectorSubcoreMesh(
      core_axis_name="c", subcore_axis_name="s", num_cores=NC, num_subcores=NS)

  out_type = jax.ShapeDtypeStruct((NC, N_PAD, width), jnp.float32)
  scratch = [
      pltpu.VMEM((NCHUNK, CHUNK), jnp.int32),
      pltpu.VMEM((NCHUNK, CHUNK), jnp.int32),
      pltpu.VMEM((CHUNK, width), jnp.float32),
      pltpu.VMEM((CHUNK, width), jnp.float32),
      pltpu.VMEM_SHARED((N_PAD, width), jnp.float32),
      pltpu.SemaphoreType.DMA,
      pltpu.SemaphoreType.DMA,
  ]
  WL = width // 16

  def body(feat_hbm, srcs_hbm, dsts_hbm, sum_hbm, sidx, didx, rows_a, rows_b,
           acc_sh, sem_a, sem_b):
    c = lax.axis_index("c")
    s = lax.axis_index("s")
    tile = c * NS + s

    pltpu.sync_copy(srcs_hbm.at[tile], sidx)
    pltpu.sync_copy(dsts_hbm.at[tile], didx)

    def zrow(i, _):
      r = i // WL
      q = (i % WL) * 16
      rows_a[r, pl.ds(q, 16)] = jnp.zeros((16,), jnp.float32)
      return _
    lax.fori_loop(0, CHUNK * WL, zrow, None)

    row0 = s * ROWS_PT
    full, part = divmod(ROWS_PT, CHUNK)
    for k in range(full):
      pltpu.sync_copy(rows_a, acc_sh.at[pl.ds(row0 + k * CHUNK, CHUNK)])
    if part:
      pltpu.sync_copy(rows_a.at[pl.ds(0, part)],
                      acc_sh.at[pl.ds(row0 + full * CHUNK, part)])
    plsc.subcore_barrier()

    pltpu.async_copy(feat_hbm.at[sidx.at[0]], rows_a, sem_a)

    def pair(i, _):
      j0 = 2 * i
      pltpu.async_copy(feat_hbm.at[sidx.at[j0 + 1]], rows_b, sem_b)
      pltpu.make_async_copy(feat_hbm.at[sidx.at[j0]], rows_a, sem_a).wait()
      pltpu.sync_copy(rows_a, acc_sh.at[didx.at[j0]], add=True)

      @pl.when(i < NPAIR - 1)
      def _():
        pltpu.async_copy(feat_hbm.at[sidx.at[j0 + 2]], rows_a, sem_a)

      pltpu.make_async_copy(feat_hbm.at[sidx.at[j0 + 1]], rows_b, sem_b).wait()
      pltpu.sync_copy(rows_b, acc_sh.at[didx.at[j0 + 1]], add=True)
      return _
    lax.fori_loop(0, NPAIR, pair, None)
    plsc.subcore_barrier()

    pltpu.sync_copy(acc_sh.at[pl.ds(row0, ROWS_PT)],
                    sum_hbm.at[c, pl.ds(row0, ROWS_PT)])

  return pl.kernel(
      body, out_type=out_type, mesh=mesh, scratch_types=scratch,
      compiler_params=pltpu.CompilerParams(use_tc_tiling_on_sc=False))


_sc_agg_feat = _sc_agg(F1)
_sc_agg_proj = _sc_agg(CP)


def _tc_fused(xp, sum0, sum1, ws1, wn1, b1, ws2, wn2, b2):
  def body(x_r, s0_r, s1_r, ws1_r, wn1_r, b1_r, ws2_r, wn2_r, b2_r, p_r, s_r):
    deg = s0_r[:, D_IN:D_IN + 1] + s1_r[:, D_IN:D_IN + 1]
    dinv = 1.0 / jnp.maximum(deg, 1.0)
    a = (s0_r[:, :D_IN] + s1_r[:, :D_IN]) * dinv
    h = (jnp.dot(x_r[...], ws1_r[...], preferred_element_type=jnp.float32)
         + jnp.dot(a, wn1_r[...], preferred_element_type=jnp.float32)
         + b1_r[...])
    h = jnp.maximum(h, 0.0)
    p_r[...] = jnp.dot(h, wn2_r[...], preferred_element_type=jnp.float32)
    s_r[...] = (jnp.dot(h, ws2_r[...], preferred_element_type=jnp.float32)
                + b2_r[...])

  row = lambda i: (i, 0)
  fix = lambda i: (0, 0)
  return pl.pallas_call(
      body,
      grid=(N_PAD // RB,),
      in_specs=[
          pl.BlockSpec((RB, D_IN), row),
          pl.BlockSpec((RB, F1), row),
          pl.BlockSpec((RB, F1), row),
          pl.BlockSpec((D_IN, D_HID), fix),
          pl.BlockSpec((D_IN, D_HID), fix),
          pl.BlockSpec((1, D_HID), fix),
          pl.BlockSpec((D_HID, CP), fix),
          pl.BlockSpec((D_HID, CP), fix),
          pl.BlockSpec((1, CP), fix),
      ],
      out_specs=[pl.BlockSpec((RB, CP), row), pl.BlockSpec((RB, CP), row)],
      out_shape=[jax.ShapeDtypeStruct((N_PAD, CP), jnp.float32)] * 2,
  )(xp, sum0, sum1, ws1, wn1, b1, ws2, wn2, b2)


def _tc_final(svec, a0, a1, d0, d1):
  def body(s_r, a0_r, a1_r, d0_r, d1_r, o_r):
    deg = d0_r[:, 0:1] + d1_r[:, 0:1]
    dinv = 1.0 / jnp.maximum(deg, 1.0)
    o_r[...] = s_r[...] + (a0_r[...] + a1_r[...]) * dinv

  row = lambda i: (i, 0)
  return pl.pallas_call(
      body,
      grid=(N_PAD // RB,),
      in_specs=[
          pl.BlockSpec((RB, CP), row),
          pl.BlockSpec((RB, CP), row),
          pl.BlockSpec((RB, CP), row),
          pl.BlockSpec((RB, 16), row),
          pl.BlockSpec((RB, 16), row),
      ],
      out_specs=pl.BlockSpec((RB, CP), row),
      out_shape=jax.ShapeDtypeStruct((N_PAD, CP), jnp.float32),
  )(svec, a0, a1, d0, d1)


@jax.jit
def kernel(x, edge_index, W_self1, W_neigh1, b1, W_self2, W_neigh2, b2):
  src = edge_index[0]
  dst = edge_index[1]
  pad_e = E_PAD - E
  srcp = jnp.concatenate(
      [src, jnp.zeros((pad_e,), jnp.int32)]).reshape(NW, NCHUNK, CHUNK)
  dstp = jnp.concatenate(
      [dst, jnp.full((pad_e,), N_PAD - 1, jnp.int32)]).reshape(NW, NCHUNK, CHUNK)
  xp = jnp.pad(x, ((0, N_PAD - N), (0, 0)))
  xa = jnp.pad(jnp.concatenate(
      [xp, jnp.ones((N_PAD, 1), jnp.float32)], axis=1),
      ((0, 0), (0, F1 - D_IN - 1)))
  ws2 = jnp.pad(W_self2, ((0, 0), (0, CP - N_CLASSES)))
  wn2 = jnp.pad(W_neigh2, ((0, 0), (0, CP - N_CLASSES)))
  b2p = jnp.pad(b2, (0, CP - N_CLASSES)).reshape(1, CP)

  sums = _sc_agg_feat(xa, srcp, dstp)
  p, svec = _tc_fused(xp, sums[0], sums[1],
                      W_self1, W_neigh1, b1.reshape(1, D_HID), ws2, wn2, b2p)
  sums2 = _sc_agg_proj(p, srcp, dstp)
  out = _tc_final(svec, sums2[0], sums2[1],
                  sums[0, :, D_IN:D_IN + 16], sums[1, :, D_IN:D_IN + 16])
  return out[:N, :N_CLASSES]


# speedup vs baseline: 8.7654x; 1.0389x over previous
import jax
import jax.numpy as jnp
from jax import lax
from jax.experimental import pallas as pl
from jax.experimental.pallas import tpu as pltpu
from jax.experimental.pallas import tpu_sc as plsc

N = 10000
E = 320000
D_IN = 128
D_HID = 256
N_CLASSES = 41
CP = 48
F1 = 144

NC = 2
NS = 16
NW = NC * NS

E_PT = 10080
E_PAD = NW * E_PT
N_PAD = 10240
ROWS_PT = N_PAD // NS
RB = 512


def _sc_agg(width, chunk):
  mesh = plsc.VectorSubcoreMesh(
      core_axis_name="c", subcore_axis_name="s", num_cores=NC, num_subcores=NS)

  NCHUNK = E_PT // chunk
  NPAIR = NCHUNK // 2
  CHUNK = chunk
  out_type = jax.ShapeDtypeStruct((NC, N_PAD, width), jnp.float32)
  scratch = [
      pltpu.VMEM((NCHUNK, CHUNK), jnp.int32),
      pltpu.VMEM((NCHUNK, CHUNK), jnp.int32),
      pltpu.VMEM((CHUNK, width), jnp.float32),
      pltpu.VMEM((CHUNK, width), jnp.float32),
      pltpu.VMEM_SHARED((N_PAD, width), jnp.float32),
      pltpu.SemaphoreType.DMA,
      pltpu.SemaphoreType.DMA,
  ]
  WL = width // 16

  def body(feat_hbm, srcs_hbm, dsts_hbm, sum_hbm, sidx, didx, rows_a, rows_b,
           acc_sh, sem_a, sem_b):
    c = lax.axis_index("c")
    s = lax.axis_index("s")
    tile = c * NS + s

    pltpu.sync_copy(srcs_hbm.at[tile], sidx)
    pltpu.sync_copy(dsts_hbm.at[tile], didx)

    def zrow(i, _):
      r = i // WL
      q = (i % WL) * 16
      rows_a[r, pl.ds(q, 16)] = jnp.zeros((16,), jnp.float32)
      return _
    lax.fori_loop(0, CHUNK * WL, zrow, None)

    row0 = s * ROWS_PT
    full, part = divmod(ROWS_PT, CHUNK)
    for k in range(full):
      pltpu.sync_copy(rows_a, acc_sh.at[pl.ds(row0 + k * CHUNK, CHUNK)])
    if part:
      pltpu.sync_copy(rows_a.at[pl.ds(0, part)],
                      acc_sh.at[pl.ds(row0 + full * CHUNK, part)])
    plsc.subcore_barrier()

    pltpu.async_copy(feat_hbm.at[sidx.at[0]], rows_a, sem_a)

    def pair(i, _):
      j0 = 2 * i
      pltpu.async_copy(feat_hbm.at[sidx.at[j0 + 1]], rows_b, sem_b)
      pltpu.make_async_copy(feat_hbm.at[sidx.at[j0]], rows_a, sem_a).wait()
      pltpu.sync_copy(rows_a, acc_sh.at[didx.at[j0]], add=True)

      @pl.when(i < NPAIR - 1)
      def _():
        pltpu.async_copy(feat_hbm.at[sidx.at[j0 + 2]], rows_a, sem_a)

      pltpu.make_async_copy(feat_hbm.at[sidx.at[j0 + 1]], rows_b, sem_b).wait()
      pltpu.sync_copy(rows_b, acc_sh.at[didx.at[j0 + 1]], add=True)
      return _
    lax.fori_loop(0, NPAIR, pair, None)
    plsc.subcore_barrier()

    pltpu.sync_copy(acc_sh.at[pl.ds(row0, ROWS_PT)],
                    sum_hbm.at[c, pl.ds(row0, ROWS_PT)])

  return pl.kernel(
      body, out_type=out_type, mesh=mesh, scratch_types=scratch,
      compiler_params=pltpu.CompilerParams(use_tc_tiling_on_sc=False))


_sc_agg_feat = _sc_agg(F1, 56)
_sc_agg_proj = _sc_agg(CP, 112)


def _tc_fused(xp, sum0, sum1, ws1, wn1, b1, ws2, wn2, b2):
  def body(x_r, s0_r, s1_r, ws1_r, wn1_r, b1_r, ws2_r, wn2_r, b2_r, p_r, s_r):
    deg = s0_r[:, D_IN:D_IN + 1] + s1_r[:, D_IN:D_IN + 1]
    dinv = 1.0 / jnp.maximum(deg, 1.0)
    a = (s0_r[:, :D_IN] + s1_r[:, :D_IN]) * dinv
    h = (jnp.dot(x_r[...], ws1_r[...], preferred_element_type=jnp.float32)
         + jnp.dot(a, wn1_r[...], preferred_element_type=jnp.float32)
         + b1_r[...])
    h = jnp.maximum(h, 0.0)
    p_r[...] = jnp.dot(h, wn2_r[...], preferred_element_type=jnp.float32)
    s_r[...] = (jnp.dot(h, ws2_r[...], preferred_element_type=jnp.float32)
                + b2_r[...])

  row = lambda i: (i, 0)
  fix = lambda i: (0, 0)
  return pl.pallas_call(
      body,
      grid=(N_PAD // RB,),
      in_specs=[
          pl.BlockSpec((RB, D_IN), row),
          pl.BlockSpec((RB, F1), row),
          pl.BlockSpec((RB, F1), row),
          pl.BlockSpec((D_IN, D_HID), fix),
          pl.BlockSpec((D_IN, D_HID), fix),
          pl.BlockSpec((1, D_HID), fix),
          pl.BlockSpec((D_HID, CP), fix),
          pl.BlockSpec((D_HID, CP), fix),
          pl.BlockSpec((1, CP), fix),
      ],
      out_specs=[pl.BlockSpec((RB, CP), row), pl.BlockSpec((RB, CP), row)],
      out_shape=[jax.ShapeDtypeStruct((N_PAD, CP), jnp.float32)] * 2,
  )(xp, sum0, sum1, ws1, wn1, b1, ws2, wn2, b2)


def _tc_final(svec, a0, a1, d0, d1):
  def body(s_r, a0_r, a1_r, d0_r, d1_r, o_r):
    deg = d0_r[:, 0:1] + d1_r[:, 0:1]
    dinv = 1.0 / jnp.maximum(deg, 1.0)
    o_r[...] = s_r[...] + (a0_r[...] + a1_r[...]) * dinv

  row = lambda i: (i, 0)
  return pl.pallas_call(
      body,
      grid=(N_PAD // RB,),
      in_specs=[
          pl.BlockSpec((RB, CP), row),
          pl.BlockSpec((RB, CP), row),
          pl.BlockSpec((RB, CP), row),
          pl.BlockSpec((RB, 16), row),
          pl.BlockSpec((RB, 16), row),
      ],
      out_specs=pl.BlockSpec((RB, CP), row),
      out_shape=jax.ShapeDtypeStruct((N_PAD, CP), jnp.float32),
  )(svec, a0, a1, d0, d1)


@jax.jit
def kernel(x, edge_index, W_self1, W_neigh1, b1, W_self2, W_neigh2, b2):
  src = edge_index[0]
  dst = edge_index[1]
  pad_e = E_PAD - E
  srcf = jnp.concatenate([src, jnp.zeros((pad_e,), jnp.int32)])
  dstf = jnp.concatenate([dst, jnp.full((pad_e,), N_PAD - 1, jnp.int32)])
  xp = jnp.pad(x, ((0, N_PAD - N), (0, 0)))
  xa = jnp.pad(jnp.concatenate(
      [xp, jnp.ones((N_PAD, 1), jnp.float32)], axis=1),
      ((0, 0), (0, F1 - D_IN - 1)))
  ws2 = jnp.pad(W_self2, ((0, 0), (0, CP - N_CLASSES)))
  wn2 = jnp.pad(W_neigh2, ((0, 0), (0, CP - N_CLASSES)))
  b2p = jnp.pad(b2, (0, CP - N_CLASSES)).reshape(1, CP)

  sums = _sc_agg_feat(xa, srcf.reshape(NW, E_PT // 56, 56),
                      dstf.reshape(NW, E_PT // 56, 56))
  p, svec = _tc_fused(xp, sums[0], sums[1],
                      W_self1, W_neigh1, b1.reshape(1, D_HID), ws2, wn2, b2p)
  sums2 = _sc_agg_proj(p, srcf.reshape(NW, E_PT // 112, 112),
                       dstf.reshape(NW, E_PT // 112, 112))
  out = _tc_final(svec, sums2[0], sums2[1],
                  sums[0, :, D_IN:D_IN + 16], sums[1, :, D_IN:D_IN + 16])
  return out[:N, :N_CLASSES]
